# Initial kernel scaffold; baseline (speedup 1.0000x reference)
import jax
import jax.numpy as jnp
from jax.experimental import pallas as pl


def kernel(x, d, sigma_uvw, beta):
    raise NotImplementedError("write your pallas kernel here")


# speedup vs baseline: 2.3928x; 2.3928x over previous
import functools

import jax
import jax.numpy as jnp
from jax import lax
from jax.experimental import pallas as pl
from jax.experimental.pallas import tpu as pltpu
from jax.experimental.pallas import tpu_sc as plsc

_SCALE = 3.0
_NP = 100
_ND = 256
_D = 32
_F = 1 + 3 * _D
_G = 16
_W = 7

_NC = 2
_NS = 16
_NW = _NC * _NS
_L = 16

_C = 16

_LN2 = 0.6931471805599453
_P5 = (0.04342890782214256, -0.40486717441919184, 1.5939013634991075,
       -3.4924942798792724, 5.046876044975866, -2.7868129538674205)

_NROWS = _NP * _NP * _NP * _F // _G


def _softplus(v):
    u = 1.0 + jnp.exp(v)
    bits = lax.bitcast_convert_type(u, jnp.int32)
    k = (bits >> 23) - 127
    mant = lax.bitcast_convert_type((bits & 0x007FFFFF) | 0x3F800000,
                                    jnp.float32)
    acc = jnp.float32(_P5[0])
    for c in _P5[1:]:
        acc = acc * mant + jnp.float32(c)
    return jnp.float32(_LN2) * (acc + k.astype(jnp.float32))


def _make_sc_call(B):
    PW = B // _NW
    NCH = PW // _C
    f32, i32 = jnp.float32, jnp.int32
    mesh = plsc.VectorSubcoreMesh(core_axis_name="c", subcore_axis_name="s")
    WC = _W * _C

    @functools.partial(
        pl.kernel,
        mesh=mesh,
        out_type=(jax.ShapeDtypeStruct((B, 3), f32),
                  jax.ShapeDtypeStruct((B,), f32)),
        compiler_params=pltpu.CompilerParams(
            needs_layout_passes=False, use_tc_tiling_on_sc=False),
        scratch_types=[
            pltpu.VMEM((_C, 3), f32),
            pltpu.VMEM((WC,), i32),
            pltpu.VMEM((_C,), i32),
            pltpu.VMEM((_C,), f32),
            pltpu.VMEM((_C,), i32),
            pltpu.VMEM((WC, _G), f32),
            pltpu.VMEM((_C, _D), f32),
            pltpu.VMEM((_C, 3), f32),
            pltpu.VMEM((_C,), f32),
            pltpu.SemaphoreType.DMA,
            pltpu.SemaphoreType.DMA,
        ],
    )
    def sc_call(x_hbm, sg_hbm, bt_hbm, color_hbm, sig_hbm,
                x_v, isg_v, ibt_v, mask_v, off_v, sg_v, bg_v, col_v, sig_v,
                sem1, sem2):
        wid = lax.axis_index("s") * _NC + lax.axis_index("c")
        zeros_i = jnp.zeros((_L,), i32)
        sg2 = sg_hbm
        bt2 = bt_hbm
        lane_masks = [lax.iota(i32, _L) == l for l in range(_L)]

        def chunk(g, carry):
            base = wid * PW + g * _C
            pltpu.sync_copy(x_hbm.at[pl.ds(base, _C)], x_v)

            half = jnp.float32(_SCALE / 2)
            step = jnp.float32(_SCALE / _NP)

            pvec = lax.iota(i32, _L)
            xi = plsc.load_gather(x_v, [pvec, zeros_i])
            yi = plsc.load_gather(x_v, [pvec, zeros_i + 1])
            zi = plsc.load_gather(x_v, [pvec, zeros_i + 2])
            inb = ((jnp.abs(xi) < half) & (jnp.abs(yi) < half)
                   & (jnp.abs(zi) < half))
            mask_v[...] = jnp.where(inb, 1.0, 0.0).astype(f32)
            ii = jnp.clip((xi / step + _NP / 2).astype(i32), 0, _NP - 1)
            jj = jnp.clip((yi / step + _NP / 2).astype(i32), 0, _NP - 1)
            kk = jnp.clip((zi / step + _NP / 2).astype(i32), 0, _NP - 1)
            w0 = ((ii * _NP + jj) * _NP + kk) * _F
            r0 = w0 >> 4
            off_v[...] = w0 & 15
            ibt_v[...] = ii * _ND + jj
            p7 = pvec * _W
            for q in range(_W):
                plsc.store_scatter(isg_v, [p7 + q], r0 + q)

            cp1 = pltpu.async_copy(sg2.at[isg_v], sg_v, sem1)
            cp2 = pltpu.async_copy(bt2.at[ibt_v], bg_v, sem2)
            cp1.wait()
            cp2.wait()

            offs = off_v[...]
            iota16 = lax.iota(i32, _L)
            acc = [jnp.zeros((_L,), f32) for _ in range(3)]
            for l in range(_L):
                wbase = l * (_W * _G) + offs[l] + iota16
                bg0 = bg_v[l, pl.ds(0, _L)]
                bg1 = bg_v[l, pl.ds(_L, _L)]
                for cc in range(3):
                    a = None
                    for h in range(2):
                        wv = wbase + (1 + cc * _D + h * _L)
                        v = plsc.load_gather(sg_v, [wv >> 4, wv & 15])
                        e = jnp.exp(v)
                        s = e / (1.0 + e)
                        w = s * (bg0 if h == 0 else bg1)
                        a = w if a is None else a + w
                    r = jnp.sum(a)
                    acc[cc] = jnp.where(lane_masks[l], r, acc[cc])

            mv = mask_v[...]
            for cc in range(3):
                plsc.store_scatter(col_v, [pvec, zeros_i + cc],
                                   acc[cc] * mv)

            w0v = pvec * (_W * _G) + offs
            sg0 = plsc.load_gather(sg_v, [w0v >> 4, w0v & 15])
            sig_v[...] = _softplus(sg0) * mv

            pltpu.sync_copy(col_v, color_hbm.at[pl.ds(base, _C)])
            pltpu.sync_copy(sig_v, sig_hbm.at[pl.ds(base, _C)])
            return carry

        lax.fori_loop(0, NCH, chunk, 0)

    return sc_call


def kernel(x, d, sigma_uvw, beta):
    del d
    B = x.shape[0]
    sg16 = sigma_uvw.reshape(_NROWS, _G)
    btt = beta.reshape(_ND * _ND, _D)
    color, sig = _make_sc_call(B)(x, sg16, btt)
    return color, sig.reshape(B, 1)


# speedup vs baseline: 3.3400x; 1.3958x over previous
import functools

import jax
import jax.numpy as jnp
from jax import lax
from jax.experimental import pallas as pl
from jax.experimental.pallas import tpu as pltpu
from jax.experimental.pallas import tpu_sc as plsc

_SCALE = 3.0
_NP = 100
_ND = 256
_D = 32
_F = 1 + 3 * _D
_FP = 112

_NC = 2
_NS = 16
_NW = _NC * _NS
_L = 16

_C = 128
_NG = _C // _L

_LN2 = 0.6931471805599453
_P5 = (0.04342890782214256, -0.40486717441919184, 1.5939013634991075,
       -3.4924942798792724, 5.046876044975866, -2.7868129538674205)


def _softplus(v):
    u = 1.0 + jnp.exp(v)
    bits = lax.bitcast_convert_type(u, jnp.int32)
    k = (bits >> 23) - 127
    mant = lax.bitcast_convert_type((bits & 0x007FFFFF) | 0x3F800000,
                                    jnp.float32)
    acc = jnp.float32(_P5[0])
    for c in _P5[1:]:
        acc = acc * mant + jnp.float32(c)
    return jnp.float32(_LN2) * (acc + k.astype(jnp.float32))


def _make_sc_call(B):
    PW = B // _NW
    NCH = PW // _C
    f32, i32 = jnp.float32, jnp.int32
    mesh = plsc.VectorSubcoreMesh(core_axis_name="c", subcore_axis_name="s")

    @functools.partial(
        pl.kernel,
        mesh=mesh,
        out_type=(jax.ShapeDtypeStruct((B, 3), f32),
                  jax.ShapeDtypeStruct((B,), f32)),
        compiler_params=pltpu.CompilerParams(
            needs_layout_passes=False, use_tc_tiling_on_sc=False),
        scratch_types=[
            pltpu.VMEM((2, 3 * _C), f32),
            pltpu.VMEM((2, _C), i32),
            pltpu.VMEM((2, _C), i32),
            pltpu.VMEM((2, _C), f32),
            pltpu.VMEM((2, _C, _FP), f32),
            pltpu.VMEM((2, _C, _D), f32),
            pltpu.VMEM((2, _C, 3), f32),
            pltpu.VMEM((2, _C), f32),
            pltpu.SemaphoreType.DMA,
            pltpu.SemaphoreType.DMA,
            pltpu.SemaphoreType.DMA,
            pltpu.SemaphoreType.DMA,
            pltpu.SemaphoreType.DMA,
            pltpu.SemaphoreType.DMA,
        ],
    )
    def sc_call(x_hbm, sg_hbm, bt_hbm, color_hbm, sig_hbm,
                x_v, isg_v, ibt_v, mask_v, sg_v, bg_v, col_v, sig_v,
                sem_x0, sem_x1, sem_g0, sem_g1, sem_o0, sem_o1):
        wid = lax.axis_index("s") * _NC + lax.axis_index("c")
        zeros_i = jnp.zeros((_L,), i32)
        iota16 = lax.iota(i32, _L)
        lane_masks = [iota16 == l for l in range(_L)]
        half = jnp.float32(_SCALE / 2)
        step = jnp.float32(_SCALE / _NP)
        wbase0 = wid * PW

        sem_x = (sem_x0, sem_x1)
        sem_g = (sem_g0, sem_g1)
        sem_o = (sem_o0, sem_o1)

        def copy_x(g, b):
            return pltpu.async_copy(
                x_hbm.at[pl.ds(3 * (wbase0 + g * _C), 3 * _C)], x_v.at[b],
                sem_x[b])

        def phase_a(g, b):
            def grp(t, c2):
                p3 = (iota16 + t * _L) * 3
                xi = plsc.load_gather(x_v.at[b], [p3])
                yi = plsc.load_gather(x_v.at[b], [p3 + 1])
                zi = plsc.load_gather(x_v.at[b], [p3 + 2])
                inb = ((jnp.abs(xi) < half) & (jnp.abs(yi) < half)
                       & (jnp.abs(zi) < half))
                mask_v.at[b][pl.ds(t * _L, _L)] = (
                    jnp.where(inb, 1.0, 0.0).astype(f32))
                ii = jnp.clip((xi / step + _NP / 2).astype(i32), 0, _NP - 1)
                jj = jnp.clip((yi / step + _NP / 2).astype(i32), 0, _NP - 1)
                kk = jnp.clip((zi / step + _NP / 2).astype(i32), 0, _NP - 1)
                isg_v.at[b][pl.ds(t * _L, _L)] = (ii * _NP + jj) * _NP + kk
                ibt_v.at[b][pl.ds(t * _L, _L)] = ii * _ND + jj
                return c2

            lax.fori_loop(0, _NG, grp, 0)

        def issue_gathers(b):
            pltpu.async_copy(sg_hbm.at[isg_v.at[b]], sg_v.at[b], sem_g[b])
            pltpu.async_copy(bt_hbm.at[ibt_v.at[b]], bg_v.at[b], sem_g[b])

        def wait_gathers(b):
            pltpu.make_async_copy(
                sg_hbm.at[isg_v.at[b]], sg_v.at[b], sem_g[b]).wait()
            pltpu.make_async_copy(
                bt_hbm.at[ibt_v.at[b]], bg_v.at[b], sem_g[b]).wait()

        def compute(g, b):
            sgb = sg_v.at[b]
            def grp(t, c2):
                pvec = iota16 + t * _L
                acc = [jnp.zeros((_L,), f32) for _ in range(3)]
                for l in range(_L):
                    p = t * _L + l
                    bg0 = bg_v[b, p, pl.ds(0, _L)]
                    bg1 = bg_v[b, p, pl.ds(_L, _L)]
                    for cc in range(3):
                        a = None
                        for h in range(2):
                            v = sg_v[b, p, pl.ds(1 + cc * _D + h * _L, _L)]
                            e = jnp.exp(v)
                            s = e / (1.0 + e)
                            w = s * (bg0 if h == 0 else bg1)
                            a = w if a is None else a + w
                        r = jnp.sum(a)
                        acc[cc] = jnp.where(lane_masks[l], r, acc[cc])
                mv = mask_v.at[b][pl.ds(t * _L, _L)]
                for cc in range(3):
                    plsc.store_scatter(col_v.at[b], [pvec, zeros_i + cc],
                                       acc[cc] * mv)
                sg0 = plsc.load_gather(sgb, [pvec, zeros_i])
                sig_v.at[b][pl.ds(t * _L, _L)] = _softplus(sg0) * mv
                return c2

            lax.fori_loop(0, _NG, grp, 0)

        def issue_outs(g, b):
            base = wbase0 + g * _C
            pltpu.async_copy(col_v.at[b], color_hbm.at[pl.ds(base, _C)],
                             sem_o[b])
            pltpu.async_copy(sig_v.at[b], sig_hbm.at[pl.ds(base, _C)],
                             sem_o[b])

        def wait_outs(g, b):
            base = wbase0 + g * _C
            pltpu.make_async_copy(col_v.at[b],
                                  color_hbm.at[pl.ds(base, _C)],
                                  sem_o[b]).wait()
            pltpu.make_async_copy(sig_v.at[b],
                                  sig_hbm.at[pl.ds(base, _C)],
                                  sem_o[b]).wait()

        copy_x(0, 0).wait()
        phase_a(0, 0)
        issue_gathers(0)
        copy_x(1, 1)

        def body2(gh, carry):
            for b in range(2):
                g = gh * 2 + b
                bn = 1 - b

                @pl.when(g + 1 < NCH)
                def _():
                    pltpu.make_async_copy(
                        x_hbm.at[pl.ds(3 * (wbase0 + (g + 1) * _C), 3 * _C)],
                        x_v.at[bn], sem_x[bn]).wait()
                    phase_a(g + 1, bn)
                    issue_gathers(bn)

                @pl.when(g + 2 < NCH)
                def _():
                    copy_x(g + 2, b)

                wait_gathers(b)

                @pl.when(g >= 2)
                def _():
                    wait_outs(g - 2, b)

                compute(g, b)
                issue_outs(g, b)
            return carry

        lax.fori_loop(0, NCH // 2, body2, 0)
        wait_outs(NCH - 2, 0)
        wait_outs(NCH - 1, 1)

    return sc_call


def kernel(x, d, sigma_uvw, beta):
    del d
    B = x.shape[0]
    sg104 = jnp.pad(sigma_uvw.reshape(_NP * _NP * _NP, _F),
                    ((0, 0), (0, _FP - _F)))
    btt = beta.reshape(_ND * _ND, _D)
    color, sig = _make_sc_call(B)(x.reshape(3 * B), sg104, btt)
    return color, sig.reshape(B, 1)
